# baseline (speedup 1.0000x reference)
import jax
import jax.numpy as jnp
from jax.experimental import pallas as pl
from jax.experimental.pallas import tpu as pltpu

M = 1024
K = 150528
N_EXP = 64
K_BLK = 7168
NUM_K = K // K_BLK


def _router_kernel(xt_ref, w_ref, b_ref, out_ref, acc_ref):
    k = pl.program_id(0)
    part = jax.lax.dot_general(
        w_ref[...].astype(jnp.bfloat16),
        xt_ref[...].astype(jnp.bfloat16),
        (((1,), (0,)), ((), ())),
        preferred_element_type=jnp.float32,
    )

    @pl.when(k == 0)
    def _init():
        acc_ref[...] = part + b_ref[...]

    @pl.when(k > 0)
    def _accum():
        acc_ref[...] += part

    @pl.when(k == NUM_K - 1)
    def _finish():
        acc = acc_ref[...]
        iota = jax.lax.broadcasted_iota(jnp.int32, acc.shape, 0)
        mx = jnp.max(acc, axis=0, keepdims=True)
        idx = jnp.min(jnp.where(acc == mx, iota, N_EXP),
                      axis=0, keepdims=True)
        out_ref[...] = idx


def kernel(x, W, b):
    xt = x.transpose(1, 2, 3, 0).reshape(K, M)
    b2 = b.reshape(N_EXP, 1)
    out = pl.pallas_call(
        _router_kernel,
        grid=(NUM_K,),
        in_specs=[
            pl.BlockSpec((K_BLK, M), lambda k: (k, 0)),
            pl.BlockSpec((N_EXP, K_BLK), lambda k: (0, k)),
            pl.BlockSpec((N_EXP, 1), lambda k: (0, 0)),
        ],
        out_specs=pl.BlockSpec((1, M), lambda k: (0, 0)),
        out_shape=jax.ShapeDtypeStruct((1, M), jnp.int32),
        scratch_shapes=[pltpu.VMEM((N_EXP, M), jnp.float32)],
        compiler_params=pltpu.CompilerParams(
            dimension_semantics=("arbitrary",),
            vmem_limit_bytes=100 * 1024 * 1024,
        ),
    )(xt, W, b2)
    return out.reshape(M)


# speedup vs baseline: 1.0335x; 1.0335x over previous
import jax
import jax.numpy as jnp
from jax.experimental import pallas as pl
from jax.experimental.pallas import tpu as pltpu

M = 1024
K = 150528
N_EXP = 64
NSTREAM = 2
K_CHUNK = 1536
K_BLK = NSTREAM * K_CHUNK
NUM_K = K // K_BLK


def _router_kernel(x0_ref, x1_ref, w_ref, b_ref, out_ref, acc_ref):
    k = pl.program_id(0)
    wb = w_ref[...].astype(jnp.bfloat16)
    part = jax.lax.dot_general(
        wb[:, :K_CHUNK],
        x0_ref[...].astype(jnp.bfloat16),
        (((1,), (0,)), ((), ())),
        preferred_element_type=jnp.float32,
    ) + jax.lax.dot_general(
        wb[:, K_CHUNK:],
        x1_ref[...].astype(jnp.bfloat16),
        (((1,), (0,)), ((), ())),
        preferred_element_type=jnp.float32,
    )

    @pl.when(k == 0)
    def _init():
        acc_ref[...] = part + b_ref[...]

    @pl.when(k > 0)
    def _accum():
        acc_ref[...] += part

    @pl.when(k == NUM_K - 1)
    def _finish():
        acc = acc_ref[...]
        iota = jax.lax.broadcasted_iota(jnp.int32, acc.shape, 0)
        mx = jnp.max(acc, axis=0, keepdims=True)
        idx = jnp.min(jnp.where(acc == mx, iota, N_EXP),
                      axis=0, keepdims=True)
        out_ref[...] = idx


def kernel(x, W, b):
    xt = x.transpose(1, 2, 3, 0).reshape(K, M)
    b2 = b.reshape(N_EXP, 1)

    def x_spec(j):
        return pl.BlockSpec((K_CHUNK, M),
                            lambda k, j=j: (k * NSTREAM + j, 0))

    out = pl.pallas_call(
        _router_kernel,
        grid=(NUM_K,),
        in_specs=[x_spec(0), x_spec(1),
                  pl.BlockSpec((N_EXP, K_BLK), lambda k: (0, k)),
                  pl.BlockSpec((N_EXP, 1), lambda k: (0, 0))],
        out_specs=pl.BlockSpec((1, M), lambda k: (0, 0)),
        out_shape=jax.ShapeDtypeStruct((1, M), jnp.int32),
        scratch_shapes=[pltpu.VMEM((N_EXP, M), jnp.float32)],
        compiler_params=pltpu.CompilerParams(
            dimension_semantics=("arbitrary",),
            vmem_limit_bytes=100 * 1024 * 1024,
        ),
    )(xt, xt, W, b2)
    return out.reshape(M)


# speedup vs baseline: 1.0345x; 1.0009x over previous
import jax
import jax.numpy as jnp
from jax.experimental import pallas as pl
from jax.experimental.pallas import tpu as pltpu

M = 1024
K = 150528
N_EXP = 64
K_BLK = 1536
NUM_K = K // K_BLK


def _router_kernel(xt_ref, w_ref, b_ref, out_ref, acc_ref):
    k = pl.program_id(0)
    part = jax.lax.dot_general(
        w_ref[...].astype(jnp.bfloat16),
        xt_ref[...].astype(jnp.bfloat16),
        (((1,), (0,)), ((), ())),
        preferred_element_type=jnp.float32,
    )

    @pl.when(k == 0)
    def _init():
        acc_ref[...] = part + b_ref[...]

    @pl.when(k > 0)
    def _accum():
        acc_ref[...] += part

    @pl.when(k == NUM_K - 1)
    def _finish():
        acc = acc_ref[...]
        iota = jax.lax.broadcasted_iota(jnp.int32, acc.shape, 0)
        mx = jnp.max(acc, axis=0, keepdims=True)
        idx = jnp.min(jnp.where(acc == mx, iota, N_EXP),
                      axis=0, keepdims=True)
        out_ref[...] = idx


def kernel(x, W, b):
    xt = x.transpose(1, 2, 3, 0).reshape(K, M)
    b2 = b.reshape(N_EXP, 1)
    out = pl.pallas_call(
        _router_kernel,
        grid=(NUM_K,),
        in_specs=[
            pl.BlockSpec((K_BLK, M), lambda k: (k, 0)),
            pl.BlockSpec((N_EXP, K_BLK), lambda k: (0, k)),
            pl.BlockSpec((N_EXP, 1), lambda k: (0, 0)),
        ],
        out_specs=pl.BlockSpec((1, M), lambda k: (0, 0)),
        out_shape=jax.ShapeDtypeStruct((1, M), jnp.int32),
        scratch_shapes=[pltpu.VMEM((N_EXP, M), jnp.float32)],
        compiler_params=pltpu.CompilerParams(
            dimension_semantics=("arbitrary",),
            vmem_limit_bytes=100 * 1024 * 1024,
        ),
    )(xt, W, b2)
    return out.reshape(M)
